# baseline (speedup 1.0000x reference)
import jax
import jax.numpy as jnp
from jax.experimental import pallas as pl
from jax.experimental.pallas import tpu as pltpu

_DIM = 128
_B = 4096


def _enqueue_body(ptr_ref, keys_ref, queue_ref, out_ref, knt_vmem, sem):
    del queue_ref
    k = keys_ref[...]
    norm = jnp.sqrt(jnp.sum(k * k, axis=1, keepdims=True))
    knt_vmem[...] = (k / jnp.maximum(norm, 1e-12)).T
    ptr = pl.multiple_of(ptr_ref[0], 512)
    pltpu.make_async_copy(
        knt_vmem, out_ref.at[:, pl.ds(ptr, _B)], sem
    ).start()
    pltpu.make_async_copy(
        knt_vmem, out_ref.at[:, pl.ds(ptr, _B)], sem
    ).wait()


def kernel(keys, queue, ptr, filled):
    keys = keys.astype(jnp.float32)
    b, dim = keys.shape
    dim2, kq = queue.shape
    assert dim == _DIM and dim2 == _DIM and b == _B

    ptr_arr = jnp.asarray(ptr, jnp.int32).reshape(1)

    new_queue = pl.pallas_call(
        _enqueue_body,
        grid=(1,),
        in_specs=[
            pl.BlockSpec(memory_space=pltpu.SMEM),
            pl.BlockSpec((b, dim), lambda j: (0, 0)),
            pl.BlockSpec(memory_space=pl.ANY),
        ],
        out_specs=pl.BlockSpec(memory_space=pl.ANY),
        out_shape=jax.ShapeDtypeStruct((dim, kq), jnp.float32),
        input_output_aliases={2: 0},
        scratch_shapes=[
            pltpu.VMEM((dim, b), jnp.float32),
            pltpu.SemaphoreType.DMA,
        ],
    )(ptr_arr, keys, queue)

    new_ptr = jnp.reshape((ptr + b) % kq, (1,)).astype(jnp.int32)
    new_filled = jnp.reshape(jnp.minimum(filled + b, kq), (1,)).astype(jnp.int32)
    return new_queue, new_ptr, new_filled
